# baseline (speedup 1.0000x reference)
import functools

import jax
import jax.numpy as jnp
from jax import lax
from jax.experimental import pallas as pl
from jax.experimental.pallas import tpu as pltpu
from jax.experimental.pallas import tpu_sc as plsc

LAMBDA_FAKE_W = 1.0
B = 256
D = 150528
NW = 32
SPT = B // NW
RPS = D // 128
CR = 168
CPS = RPS // CR
NCH = SPT * CPS
UNROLL = 8


def _chunk_sum(buf_r, buf_x):
    def body(i, acc):
        v = []
        for k in range(UNROLL):
            rv = buf_r[i, pl.ds(k * 16, 16)]
            xv = buf_x[i, pl.ds(k * 16, 16)]
            v.append(jnp.abs(rv - xv))
        t = ((v[0] + v[1]) + (v[2] + v[3])) + ((v[4] + v[5]) + (v[6] + v[7]))
        return acc + t
    return lax.fori_loop(0, CR, body, jnp.zeros((16,), jnp.float32))


def _sc_partials(r_hbm, x_hbm, y_hbm, out_hbm, rbuf, xbuf, ybuf, obuf,
                 rsem, xsem, ysem, osem):
    cid = lax.axis_index("c")
    sid = lax.axis_index("s")
    wid = sid * 2 + cid
    row0 = wid * SPT * RPS

    pltpu.async_copy(y_hbm.at[pl.ds(wid * SPT, SPT), :], ybuf, ysem).wait()

    handles = {}

    def start(c, slot):
        handles[slot] = (
            pltpu.async_copy(r_hbm.at[pl.ds(row0 + c * CR, CR), :],
                             rbuf.at[slot], rsem.at[slot]),
            pltpu.async_copy(x_hbm.at[pl.ds(row0 + c * CR, CR), :],
                             xbuf.at[slot], xsem.at[slot]),
        )

    def wait(slot):
        hr, hx = handles[slot]
        hr.wait()
        hx.wait()

    acc_real = jnp.zeros((16,), jnp.float32)
    acc_all = jnp.zeros((16,), jnp.float32)
    acc_cnt = jnp.zeros((16,), jnp.float32)

    start(0, 0)
    for c in range(NCH):
        if c + 1 < NCH:
            start(c + 1, (c + 1) % 2)
        wait(c % 2)
        cs = _chunk_sum(rbuf.at[c % 2], xbuf.at[c % 2])
        yrow = ybuf[c // CPS, pl.ds(0, 16)]
        acc_all = acc_all + cs
        acc_real = acc_real + cs * yrow
        if c % CPS == 0:
            acc_cnt = acc_cnt + yrow

    zeros = jnp.zeros((16,), jnp.float32)
    for row, acc in ((0, acc_real), (1, acc_all), (2, acc_cnt)):
        obuf[row, pl.ds(0, 16)] = acc
        for k in range(1, 8):
            obuf[row, pl.ds(k * 16, 16)] = zeros
    for row in range(3, 8):
        for k in range(8):
            obuf[row, pl.ds(k * 16, 16)] = zeros
    pltpu.async_copy(obuf, out_hbm.at[pl.ds(wid * 8, 8), :], osem).wait()


_sc_call = functools.partial(
    pl.kernel,
    out_type=jax.ShapeDtypeStruct((NW * 8, 128), jnp.float32),
    mesh=plsc.VectorSubcoreMesh(core_axis_name="c", subcore_axis_name="s"),
    compiler_params=pltpu.CompilerParams(use_tc_tiling_on_sc=True),
    scratch_types=[
        pltpu.VMEM((2, CR, 128), jnp.float32),
        pltpu.VMEM((2, CR, 128), jnp.float32),
        pltpu.VMEM((SPT, 128), jnp.float32),
        pltpu.VMEM((8, 128), jnp.float32),
        pltpu.SemaphoreType.DMA((2,)),
        pltpu.SemaphoreType.DMA((2,)),
        pltpu.SemaphoreType.DMA,
        pltpu.SemaphoreType.DMA,
    ],
)(_sc_partials)


def kernel(recons, x, y):
    rc = recons.reshape(B * RPS, 128)
    xc = x.reshape(B * RPS, 128)
    y128 = jnp.broadcast_to(y.astype(jnp.float32)[:, None], (B, 128))

    parts = _sc_call(rc, xc, y128).reshape(NW, 8, 128)
    sum_real = jnp.sum(parts[:, 0, :])
    sum_all = jnp.sum(parts[:, 1, :])
    n_real = jnp.sum(parts[:, 2, :]) / 16.0
    n_fake = B - n_real
    sum_fake = sum_all - sum_real
    loss_real = jnp.where(n_real > 0, sum_real / (n_real * D), 0.0)
    loss_fake = jnp.where(n_fake > 0, sum_fake / (n_fake * D), 0.0)
    return loss_real - LAMBDA_FAKE_W * loss_fake


# speedup vs baseline: 2.0908x; 2.0908x over previous
import jax
import jax.numpy as jnp
from jax.experimental import pallas as pl
from jax.experimental.pallas import tpu as pltpu

LAMBDA_FAKE_W = 1.0
B = 256
D = 150528
RB = 8
NSTEPS = B // RB
NSLICE = 4
SLW = D // NSLICE


def _loss_kernel(y_ref, *refs):
    o_ref, acc_ref = refs[-2], refs[-1]
    in_refs = refs[:-2]
    step = pl.program_id(0)

    @pl.when(step == 0)
    def _init():
        acc_ref[0] = 0.0
        acc_ref[1] = 0.0
        acc_ref[2] = 0.0

    s = jnp.zeros((RB, 1), jnp.float32)
    for k in range(NSLICE):
        r_ref = in_refs[k]
        x_ref = in_refs[NSLICE + k]
        d = jnp.abs(r_ref[...] - x_ref[...])
        s = s + jnp.sum(d, axis=1, keepdims=True)
    yv = y_ref[...].astype(jnp.float32)
    acc_ref[0] += jnp.sum(s * yv)
    acc_ref[1] += jnp.sum(s)
    acc_ref[2] += jnp.sum(yv)

    @pl.when(step == NSTEPS - 1)
    def _finalize():
        n_real = acc_ref[2]
        n_fake = B - n_real
        sum_real = acc_ref[0]
        sum_fake = acc_ref[1] - sum_real
        loss_real = jnp.where(n_real > 0, sum_real / (n_real * D), 0.0)
        loss_fake = jnp.where(n_fake > 0, sum_fake / (n_fake * D), 0.0)
        o_ref[...] = (loss_real - LAMBDA_FAKE_W * loss_fake).reshape(1, 1)


def kernel(recons, x, y):
    r2 = recons.reshape(B, D)
    x2 = x.reshape(B, D)
    y2 = y.astype(jnp.float32).reshape(B, 1)

    operands = [r2] * NSLICE + [x2] * NSLICE

    def _mk_spec(k):
        return pl.BlockSpec((RB, SLW), lambda i, _k=k: (i, _k))

    big_specs = [_mk_spec(k) for k in range(NSLICE)] * 2
    out = pl.pallas_call(
        _loss_kernel,
        grid=(NSTEPS,),
        in_specs=[pl.BlockSpec((RB, 1), lambda i: (i, 0))] + big_specs,
        out_specs=pl.BlockSpec((1, 1), lambda i: (0, 0)),
        out_shape=jax.ShapeDtypeStruct((1, 1), jnp.float32),
        scratch_shapes=[pltpu.SMEM((3,), jnp.float32)],
        compiler_params=pltpu.CompilerParams(
            dimension_semantics=("arbitrary",),
        ),
    )(y2, *operands)
    return out.reshape(())
